# baseline (speedup 1.0000x reference)
import functools

import jax
import jax.numpy as jnp
from jax import lax
from jax.experimental import pallas as pl
from jax.experimental.pallas import tpu as pltpu
from jax.experimental.pallas import tpu_sc as plsc

_N = 10000
_E = 320000
_DIN = 128
_H = 64

_NC = 2
_NS = 16
_L = 16
_NW = _NC * _NS
_NP = 10240
_SL = _NP // _NS
_EPT = _E // _NW
_K = 128
_EP = 2560 * _K
_NCH = _EP // _K // _NW
_ER = _EP // _K
_PF = 2
_NB = 4

_mesh = plsc.VectorSubcoreMesh(core_axis_name="core", subcore_axis_name="subcore")


@functools.partial(
    pl.kernel,
    out_type=jax.ShapeDtypeStruct((_NC, _NP), jnp.float32),
    mesh=_mesh,
    compiler_params=pltpu.CompilerParams(use_tc_tiling_on_sc=False),
    scratch_types=[
        pltpu.VMEM_SHARED((_NP,), jnp.float32),
        pltpu.VMEM((_NCH, _K), jnp.int32),
        pltpu.VMEM((_K,), jnp.float32),
        pltpu.VMEM((_SL,), jnp.float32),
        pltpu.SemaphoreType.DMA,
    ],
)
def _sc_degree(dst2_hbm, cnt_hbm, cnt_sh, idx_v, ones_v, zb_v, sem):
    cid = lax.axis_index("core")
    sid = lax.axis_index("subcore")
    wid = cid * _NS + sid
    row = sid * _SL

    @pl.loop(0, _SL, step=_L)
    def _(j):
        zb_v[pl.ds(j, _L)] = jnp.zeros((_L,), jnp.float32)

    pltpu.sync_copy(zb_v, cnt_sh.at[pl.ds(row, _SL)])
    pltpu.sync_copy(dst2_hbm.at[pl.ds(wid * _NCH, _NCH)], idx_v)

    @pl.loop(0, _K, step=_L)
    def _(j):
        ones_v[pl.ds(j, _L)] = jnp.ones((_L,), jnp.float32)

    plsc.subcore_barrier()

    @pl.loop(0, _NCH, step=5)
    def _(i):
        for k in range(5):
            pltpu.async_copy(ones_v, cnt_sh.at[idx_v.at[i + k]], sem, add=True)
        for k in range(5):
            pltpu.make_async_copy(
                ones_v, cnt_sh.at[idx_v.at[i + k]], sem).wait()

    plsc.subcore_barrier()
    pltpu.sync_copy(cnt_sh.at[pl.ds(row, _SL)], cnt_hbm.at[cid, pl.ds(row, _SL)])


@functools.partial(
    pl.kernel,
    out_type=(
        jax.ShapeDtypeStruct((_NC, _NP, _H), jnp.float32),
        jax.ShapeDtypeStruct((_NC, _NP), jnp.float32),
    ),
    mesh=_mesh,
    compiler_params=pltpu.CompilerParams(
        needs_layout_passes=False, use_tc_tiling_on_sc=False),
    scratch_types=[
        pltpu.VMEM_SHARED((_NP, _H), jnp.float32),
        pltpu.VMEM_SHARED((_NP,), jnp.float32),
        pltpu.VMEM_SHARED((_NP,), jnp.float32),
        pltpu.VMEM((_NP,), jnp.float32),
        pltpu.VMEM((_NCH, _K), jnp.int32),
        pltpu.VMEM((_NCH, _K), jnp.int32),
        pltpu.VMEM((_NB, _K, _H), jnp.float32),
        pltpu.VMEM((_NB, _K), jnp.float32),
        pltpu.VMEM((_SL,), jnp.float32),
        pltpu.VMEM((_SL,), jnp.float32),
        pltpu.VMEM((_SL,), jnp.float32),
        pltpu.VMEM((128, _H), jnp.float32),
        pltpu.SemaphoreType.DMA((_NB,)),
        pltpu.SemaphoreType.DMA((_NB,)),
        pltpu.SemaphoreType.DMA((_NB,)),
    ],
)
def _sc_aggregate(src2_hbm, dst2_hbm, y1_hbm, cntp_hbm,
                  seg_hbm, c_hbm,
                  seg_sh, c_sh, inv_sh, inv_v, src_v, dst_v, rows_v, vals_v,
                  cnt0_v, cnt1_v, invs_v, segb_v, gsem, rsem, vsem):
    cid = lax.axis_index("core")
    sid = lax.axis_index("subcore")
    wid = cid * _NS + sid
    row = sid * _SL

    @pl.loop(0, 128)
    def _(r):
        for q in range(_H // _L):
            segb_v[r, pl.ds(q * _L, _L)] = jnp.zeros((_L,), jnp.float32)

    @pl.loop(0, _SL, step=128)
    def _(r0):
        pltpu.sync_copy(segb_v, seg_sh.at[pl.ds(row + r0, 128)])

    @pl.loop(0, _SL, step=_L)
    def _(j):
        invs_v[pl.ds(j, _L)] = jnp.zeros((_L,), jnp.float32)

    pltpu.sync_copy(invs_v, c_sh.at[pl.ds(row, _SL)])

    pltpu.sync_copy(src2_hbm.at[pl.ds(wid * _NCH, _NCH)], src_v)
    pltpu.sync_copy(dst2_hbm.at[pl.ds(wid * _NCH, _NCH)], dst_v)

    pltpu.sync_copy(cntp_hbm.at[0, pl.ds(row, _SL)], cnt0_v)
    pltpu.sync_copy(cntp_hbm.at[1, pl.ds(row, _SL)], cnt1_v)

    @pl.loop(0, _SL, step=_L)
    def _(i):
        a = cnt0_v[pl.ds(i, _L)] + cnt1_v[pl.ds(i, _L)]
        invs_v[pl.ds(i, _L)] = 1.0 / jnp.maximum(a, 1.0)

    pltpu.sync_copy(invs_v, inv_sh.at[pl.ds(row, _SL)])
    plsc.subcore_barrier()

    pltpu.sync_copy(inv_sh, inv_v)

    def _gather(i, b):
        pltpu.async_copy(y1_hbm.at[src_v.at[i]], rows_v.at[b], gsem.at[b])

    def _wait_gather(i, b):
        pltpu.make_async_copy(
            y1_hbm.at[src_v.at[i]], rows_v.at[b], gsem.at[b]).wait()

    def _wait_rscat(i, b):
        pltpu.make_async_copy(
            rows_v.at[b], seg_sh.at[dst_v.at[i]], rsem.at[b]).wait()

    def _wait_vscat(i, b):
        pltpu.make_async_copy(
            vals_v.at[b], c_sh.at[src_v.at[i]], vsem.at[b]).wait()

    def _vals(i, b):
        for j in range(_K // _L):
            iv = dst_v.at[i][pl.ds(j * _L, _L)]
            vals_v[b, pl.ds(j * _L, _L)] = plsc.load_gather(inv_v, [iv])
        pltpu.async_copy(vals_v.at[b], c_sh.at[src_v.at[i]], vsem.at[b],
                         add=True)

    def _rscat(i, b):
        pltpu.async_copy(rows_v.at[b], seg_sh.at[dst_v.at[i]], rsem.at[b],
                         add=True)

    for b in range(_PF):
        _gather(b, b)

    _NG = _NCH // _NB

    @pl.loop(0, _NG)
    def _(g):
        for b in range(_NB):
            i = g * _NB + b
            _wait_gather(i, b)
            _rscat(i, b)

            @pl.when(g > 0)
            def _():
                _wait_vscat(i - _NB, b)

            _vals(i, b)

            b2 = (b + _PF) % _NB
            if b >= _PF:

                @pl.when(i + _PF < _NCH)
                def _():
                    _wait_rscat(i - _PF, b2)
                    _gather(i + _PF, b2)

            else:

                @pl.when(g > 0)
                def _():
                    _wait_rscat(i - _PF, b2)

                @pl.when(i + _PF < _NCH)
                def _():
                    _gather(i + _PF, b2)

    for i in range(_NCH - _NB, _NCH):
        b = i % _NB
        _wait_rscat(i, b)
        _wait_vscat(i, b)

    plsc.subcore_barrier()

    @pl.loop(0, _SL, step=128)
    def _(r0):
        pltpu.sync_copy(seg_sh.at[pl.ds(row + r0, 128)], segb_v)

        @pl.loop(0, 128)
        def _(r):
            s = plsc.load_gather(invs_v, [jnp.full((_L,), r0 + r, jnp.int32)])
            for q in range(_H // _L):
                segb_v[r, pl.ds(q * _L, _L)] = segb_v[r, pl.ds(q * _L, _L)] * s

        pltpu.sync_copy(segb_v, seg_hbm.at[cid, pl.ds(row + r0, 128)])
    pltpu.sync_copy(c_sh.at[pl.ds(row, _SL)], c_hbm.at[cid, pl.ds(row, _SL)])


def _tc_pre_body(x_ref, wl_ref, wr_ref, y1_ref, r1_ref):
    y1 = jnp.dot(x_ref[...], wl_ref[...], preferred_element_type=jnp.float32)
    r1 = jnp.dot(x_ref[...], wr_ref[...], preferred_element_type=jnp.float32)
    y1_ref[pl.ds(0, _N), :] = y1
    r1_ref[pl.ds(0, _N), :] = r1
    pad = jnp.zeros((_NP - _N, _H), jnp.float32)
    y1_ref[pl.ds(_N, _NP - _N), :] = pad
    r1_ref[pl.ds(_N, _NP - _N), :] = pad


_tc_pre = pl.pallas_call(
    _tc_pre_body,
    out_shape=(
        jax.ShapeDtypeStruct((_NP, _H), jnp.float32),
        jax.ShapeDtypeStruct((_NP, _H), jnp.float32),
    ),
)


def _tc_post_body(segp_ref, cp_ref, r1_ref,
                  b1_ref, g_ref, bt_ref, w2l_ref, w2r_ref, b2_ref, out_ref):
    mask = (lax.broadcasted_iota(jnp.int32, (_NP, 1), 0) < _N).astype(
        jnp.float32)
    z = segp_ref[0] + segp_ref[1] + r1_ref[...] + b1_ref[...]
    mean = jnp.sum(z * mask, axis=0, keepdims=True) * (1.0 / _N)
    zc = z - mean
    var = jnp.sum(zc * zc * mask, axis=0, keepdims=True) * (1.0 / _N)
    h = g_ref[...] * zc * lax.rsqrt(var + 1e-5) + bt_ref[...]
    hm = jnp.maximum(h, 0.0) * mask
    cc = cp_ref[0:1, :] + cp_ref[1:2, :]
    s1 = jnp.dot(cc, hm, preferred_element_type=jnp.float32) * (1.0 / _N)
    s2 = jnp.sum(hm, axis=0, keepdims=True) * (1.0 / _N)
    out_ref[...] = (
        jnp.dot(s1, w2l_ref[...], preferred_element_type=jnp.float32)
        + jnp.dot(s2, w2r_ref[...], preferred_element_type=jnp.float32)
        + b2_ref[...]
    )


_tc_post = pl.pallas_call(
    _tc_post_body,
    out_shape=jax.ShapeDtypeStruct((1, _H), jnp.float32),
)


def kernel(x, edge_index, W1l, b1, W1r, gamma, beta, W2l, b2, W2r):
    pad = jnp.tile(jnp.array([[_N], [_NP - 1]], jnp.int32), (1, _EP - _E))
    e2 = jnp.concatenate([edge_index, pad], axis=1)
    src2 = e2[0].reshape(_ER, _K)
    dst2 = e2[1].reshape(_ER, _K)

    y1, r1 = _tc_pre(x, W1l, W1r)
    cntp = _sc_degree(dst2)
    segp, cp = _sc_aggregate(src2, dst2, y1, cntp)

    return _tc_post(
        segp, cp, r1,
        b1[None, :], gamma[None, :], beta[None, :],
        W2l, W2r, b2[None, :],
    )


# speedup vs baseline: 2.2248x; 2.2248x over previous
import functools

import jax
import jax.numpy as jnp
from jax import lax
from jax.experimental import pallas as pl
from jax.experimental.pallas import tpu as pltpu
from jax.experimental.pallas import tpu_sc as plsc

_N = 10000
_E = 320000
_DIN = 128
_H = 64

_NC = 2
_NS = 16
_L = 16
_NW = _NC * _NS
_NP = 10240
_SL = _NP // _NS
_EPT = _E // _NW
_K = 128
_EP = 2560 * _K
_NCH = _EP // _K // _NW
_ER = _EP // _K
_PF = 2
_NB = 4

_mesh = plsc.VectorSubcoreMesh(core_axis_name="core", subcore_axis_name="subcore")


@functools.partial(
    pl.kernel,
    out_type=jax.ShapeDtypeStruct((_NC, _NP), jnp.float32),
    mesh=_mesh,
    compiler_params=pltpu.CompilerParams(use_tc_tiling_on_sc=False),
    scratch_types=[
        pltpu.VMEM_SHARED((_NP,), jnp.float32),
        pltpu.VMEM((_NCH, _K), jnp.int32),
        pltpu.VMEM((_K,), jnp.float32),
        pltpu.VMEM((_SL,), jnp.float32),
        pltpu.SemaphoreType.DMA,
    ],
)
def _sc_degree(dst2_hbm, cnt_hbm, cnt_sh, idx_v, ones_v, zb_v, sem):
    cid = lax.axis_index("core")
    sid = lax.axis_index("subcore")
    wid = cid * _NS + sid
    row = sid * _SL

    @pl.loop(0, _SL, step=_L)
    def _(j):
        zb_v[pl.ds(j, _L)] = jnp.zeros((_L,), jnp.float32)

    pltpu.sync_copy(zb_v, cnt_sh.at[pl.ds(row, _SL)])
    pltpu.sync_copy(dst2_hbm.at[pl.ds(wid * _NCH, _NCH)], idx_v)

    @pl.loop(0, _K, step=_L)
    def _(j):
        ones_v[pl.ds(j, _L)] = jnp.ones((_L,), jnp.float32)

    plsc.subcore_barrier()

    @pl.loop(0, _NCH, step=5)
    def _(i):
        for k in range(5):
            pltpu.async_copy(ones_v, cnt_sh.at[idx_v.at[i + k]], sem, add=True)
        for k in range(5):
            pltpu.make_async_copy(
                ones_v, cnt_sh.at[idx_v.at[i + k]], sem).wait()

    plsc.subcore_barrier()
    pltpu.sync_copy(cnt_sh.at[pl.ds(row, _SL)], cnt_hbm.at[cid, pl.ds(row, _SL)])


@functools.partial(
    pl.kernel,
    out_type=(
        jax.ShapeDtypeStruct((_NC, _NP, _H), jnp.float32),
        jax.ShapeDtypeStruct((_NC, _NP), jnp.float32),
    ),
    mesh=_mesh,
    compiler_params=pltpu.CompilerParams(
        needs_layout_passes=False, use_tc_tiling_on_sc=False),
    scratch_types=[
        pltpu.VMEM_SHARED((_NP, _H), jnp.float32),
        pltpu.VMEM_SHARED((_NP,), jnp.float32),
        pltpu.VMEM_SHARED((_NP,), jnp.float32),
        pltpu.VMEM((_NP,), jnp.float32),
        pltpu.VMEM((_NCH, _K), jnp.int32),
        pltpu.VMEM((_NCH, _K), jnp.int32),
        pltpu.VMEM((_NB, _K, _H), jnp.float32),
        pltpu.VMEM((_NB, _K), jnp.float32),
        pltpu.VMEM((_SL,), jnp.float32),
        pltpu.VMEM((_SL,), jnp.float32),
        pltpu.VMEM((_SL,), jnp.float32),
        pltpu.VMEM((128, _H), jnp.float32),
        pltpu.SemaphoreType.DMA((_NB,)),
        pltpu.SemaphoreType.DMA((_NB,)),
        pltpu.SemaphoreType.DMA((_NB,)),
    ],
)
def _sc_aggregate(src2_hbm, dst2_hbm, y1_hbm, cntp_hbm,
                  seg_hbm, c_hbm,
                  seg_sh, c_sh, inv_sh, inv_v, src_v, dst_v, rows_v, vals_v,
                  cnt0_v, cnt1_v, invs_v, segb_v, gsem, rsem, vsem):
    cid = lax.axis_index("core")
    sid = lax.axis_index("subcore")
    wid = cid * _NS + sid
    row = sid * _SL

    @pl.loop(0, 128)
    def _(r):
        for q in range(_H // _L):
            segb_v[r, pl.ds(q * _L, _L)] = jnp.zeros((_L,), jnp.float32)

    @pl.loop(0, _SL, step=128)
    def _(r0):
        pltpu.sync_copy(segb_v, seg_sh.at[pl.ds(row + r0, 128)])

    @pl.loop(0, _SL, step=_L)
    def _(j):
        invs_v[pl.ds(j, _L)] = jnp.zeros((_L,), jnp.float32)

    pltpu.sync_copy(invs_v, c_sh.at[pl.ds(row, _SL)])

    pltpu.sync_copy(src2_hbm.at[pl.ds(wid * _NCH, _NCH)], src_v)
    pltpu.sync_copy(dst2_hbm.at[pl.ds(wid * _NCH, _NCH)], dst_v)

    pltpu.sync_copy(cntp_hbm.at[0, pl.ds(row, _SL)], cnt0_v)
    pltpu.sync_copy(cntp_hbm.at[1, pl.ds(row, _SL)], cnt1_v)

    @pl.loop(0, _SL, step=_L)
    def _(i):
        a = cnt0_v[pl.ds(i, _L)] + cnt1_v[pl.ds(i, _L)]
        invs_v[pl.ds(i, _L)] = 1.0 / jnp.maximum(a, 1.0)

    pltpu.sync_copy(invs_v, inv_sh.at[pl.ds(row, _SL)])
    plsc.subcore_barrier()

    pltpu.sync_copy(inv_sh, inv_v)

    def _gather(i, b):
        pltpu.async_copy(y1_hbm.at[src_v.at[i]], rows_v.at[b], gsem.at[b])

    def _wait_gather(i, b):
        pltpu.make_async_copy(
            y1_hbm.at[src_v.at[i]], rows_v.at[b], gsem.at[b]).wait()

    def _wait_rscat(i, b):
        pltpu.make_async_copy(
            rows_v.at[b], seg_sh.at[dst_v.at[i]], rsem.at[b]).wait()

    def _wait_vscat(i, b):
        pltpu.make_async_copy(
            vals_v.at[b], c_sh.at[src_v.at[i]], vsem.at[b]).wait()

    def _vals(i, b):
        for j in range(_K // _L):
            iv = dst_v.at[i][pl.ds(j * _L, _L)]
            vals_v[b, pl.ds(j * _L, _L)] = plsc.load_gather(inv_v, [iv])
        pltpu.async_copy(vals_v.at[b], c_sh.at[src_v.at[i]], vsem.at[b],
                         add=True)

    def _rscat(i, b):
        pltpu.async_copy(rows_v.at[b], seg_sh.at[dst_v.at[i]], rsem.at[b],
                         add=True)

    for b in range(_PF):
        _gather(b, b)

    _NG = _NCH // _NB

    @pl.loop(0, _NG)
    def _(g):
        for b in range(_NB):
            i = g * _NB + b
            _wait_gather(i, b)
            _rscat(i, b)

            @pl.when(g > 0)
            def _():
                _wait_vscat(i - _NB, b)

            _vals(i, b)

            b2 = (b + _PF) % _NB
            if b >= _PF:

                @pl.when(i + _PF < _NCH)
                def _():
                    _wait_rscat(i - _PF, b2)
                    _gather(i + _PF, b2)

            else:

                @pl.when(g > 0)
                def _():
                    _wait_rscat(i - _PF, b2)

                @pl.when(i + _PF < _NCH)
                def _():
                    _gather(i + _PF, b2)

    for i in range(_NCH - _NB, _NCH):
        b = i % _NB
        _wait_rscat(i, b)
        _wait_vscat(i, b)

    plsc.subcore_barrier()

    @pl.loop(0, _SL, step=128)
    def _(r0):
        pltpu.sync_copy(seg_sh.at[pl.ds(row + r0, 128)], segb_v)

        @pl.loop(0, 128)
        def _(r):
            s = plsc.load_gather(invs_v, [jnp.full((_L,), r0 + r, jnp.int32)])
            for q in range(_H // _L):
                segb_v[r, pl.ds(q * _L, _L)] = segb_v[r, pl.ds(q * _L, _L)] * s

        pltpu.sync_copy(segb_v, seg_hbm.at[cid, pl.ds(row + r0, 128)])
    pltpu.sync_copy(c_sh.at[pl.ds(row, _SL)], c_hbm.at[cid, pl.ds(row, _SL)])


def _tc_pre_body(x_ref, wl_ref, wr_ref, y1_ref, r1_ref):
    y1 = jnp.dot(x_ref[...], wl_ref[...], preferred_element_type=jnp.float32)
    r1 = jnp.dot(x_ref[...], wr_ref[...], preferred_element_type=jnp.float32)
    y1_ref[pl.ds(0, _N), :] = y1
    r1_ref[pl.ds(0, _N), :] = r1
    pad = jnp.zeros((_NP - _N, _H), jnp.float32)
    y1_ref[pl.ds(_N, _NP - _N), :] = pad
    r1_ref[pl.ds(_N, _NP - _N), :] = pad


_tc_pre = pl.pallas_call(
    _tc_pre_body,
    out_shape=(
        jax.ShapeDtypeStruct((_NP, _H), jnp.float32),
        jax.ShapeDtypeStruct((_NP, _H), jnp.float32),
    ),
)


def _tc_post_body(segp_ref, cp_ref, r1_ref,
                  b1_ref, g_ref, bt_ref, w2l_ref, w2r_ref, b2_ref, out_ref):
    mask = (lax.broadcasted_iota(jnp.int32, (_NP, 1), 0) < _N).astype(
        jnp.float32)
    z = segp_ref[0] + segp_ref[1] + r1_ref[...] + b1_ref[...]
    mean = jnp.sum(z * mask, axis=0, keepdims=True) * (1.0 / _N)
    zc = z - mean
    var = jnp.sum(zc * zc * mask, axis=0, keepdims=True) * (1.0 / _N)
    h = g_ref[...] * zc * lax.rsqrt(var + 1e-5) + bt_ref[...]
    hm = jnp.maximum(h, 0.0) * mask
    cc = cp_ref[0:1, :] + cp_ref[1:2, :]
    s1 = jnp.dot(cc, hm, preferred_element_type=jnp.float32) * (1.0 / _N)
    s2 = jnp.sum(hm, axis=0, keepdims=True) * (1.0 / _N)
    out_ref[...] = (
        jnp.dot(s1, w2l_ref[...], preferred_element_type=jnp.float32)
        + jnp.dot(s2, w2r_ref[...], preferred_element_type=jnp.float32)
        + b2_ref[...]
    )


_tc_post = pl.pallas_call(
    _tc_post_body,
    out_shape=jax.ShapeDtypeStruct((1, _H), jnp.float32),
)


def kernel(x, edge_index, W1l, b1, W1r, gamma, beta, W2l, b2, W2r):
    cyc = _N + jnp.arange(_EP - _E, dtype=jnp.int32) % (_NP - _N)
    e2 = jnp.concatenate([edge_index, jnp.stack([cyc, cyc])], axis=1)
    src2 = e2[0].reshape(_ER, _K)
    dst2 = e2[1].reshape(_ER, _K)

    y1, r1 = _tc_pre(x, W1l, W1r)
    cntp = _sc_degree(dst2)
    segp, cp = _sc_aggregate(src2, dst2, y1, cntp)

    return _tc_post(
        segp, cp, r1,
        b1[None, :], gamma[None, :], beta[None, :],
        W2l, W2r, b2[None, :],
    )


# speedup vs baseline: 2.2732x; 1.0217x over previous
import functools

import jax
import jax.numpy as jnp
from jax import lax
from jax.experimental import pallas as pl
from jax.experimental.pallas import tpu as pltpu
from jax.experimental.pallas import tpu_sc as plsc

_N = 10000
_E = 320000
_DIN = 128
_H = 64

_NC = 2
_NS = 16
_L = 16
_NW = _NC * _NS
_NP = 10240
_SL = _NP // _NS
_EPT = _E // _NW
_K = 128
_EP = 2560 * _K
_NCH = _EP // _K // _NW
_ER = _EP // _K
_PF = 2
_NB = 4

_mesh = plsc.VectorSubcoreMesh(core_axis_name="core", subcore_axis_name="subcore")


@functools.partial(
    pl.kernel,
    out_type=jax.ShapeDtypeStruct((_NC, _NP), jnp.float32),
    mesh=_mesh,
    compiler_params=pltpu.CompilerParams(use_tc_tiling_on_sc=False),
    scratch_types=[
        pltpu.VMEM_SHARED((_NP,), jnp.float32),
        pltpu.VMEM((_NCH, _K), jnp.int32),
        pltpu.VMEM((_K,), jnp.float32),
        pltpu.VMEM((_SL,), jnp.float32),
        pltpu.SemaphoreType.DMA,
    ],
)
def _sc_degree(dst2_hbm, cnt_hbm, cnt_sh, idx_v, ones_v, zb_v, sem):
    cid = lax.axis_index("core")
    sid = lax.axis_index("subcore")
    wid = cid * _NS + sid
    row = sid * _SL

    @pl.loop(0, _SL, step=_L)
    def _(j):
        zb_v[pl.ds(j, _L)] = jnp.zeros((_L,), jnp.float32)

    pltpu.sync_copy(zb_v, cnt_sh.at[pl.ds(row, _SL)])
    pltpu.sync_copy(dst2_hbm.at[pl.ds(wid * _NCH, _NCH)], idx_v)

    @pl.loop(0, _K, step=_L)
    def _(j):
        ones_v[pl.ds(j, _L)] = jnp.ones((_L,), jnp.float32)

    plsc.subcore_barrier()

    @pl.loop(0, _NCH, step=5)
    def _(i):
        for k in range(5):
            pltpu.async_copy(ones_v, cnt_sh.at[idx_v.at[i + k]], sem, add=True)
        for k in range(5):
            pltpu.make_async_copy(
                ones_v, cnt_sh.at[idx_v.at[i + k]], sem).wait()

    plsc.subcore_barrier()
    pltpu.sync_copy(cnt_sh.at[pl.ds(row, _SL)], cnt_hbm.at[cid, pl.ds(row, _SL)])


@functools.partial(
    pl.kernel,
    out_type=(
        jax.ShapeDtypeStruct((_NC, _NP // 2, 2 * _H), jnp.float32),
        jax.ShapeDtypeStruct((_NC, _NP // 2, 2 * _H), jnp.float32),
    ),
    mesh=_mesh,
    compiler_params=pltpu.CompilerParams(
        needs_layout_passes=False, use_tc_tiling_on_sc=False),
    scratch_types=[
        pltpu.VMEM_SHARED((_NP, _H), jnp.float32),
        pltpu.VMEM_SHARED((_NP,), jnp.float32),
        pltpu.VMEM_SHARED((_NP,), jnp.float32),
        pltpu.VMEM((_NP,), jnp.float32),
        pltpu.VMEM((_NCH, _K), jnp.int32),
        pltpu.VMEM((_NCH, _K), jnp.int32),
        pltpu.VMEM((_NB, _K, _H), jnp.float32),
        pltpu.VMEM((_NB, _K), jnp.float32),
        pltpu.VMEM((_SL,), jnp.float32),
        pltpu.VMEM((_SL,), jnp.float32),
        pltpu.VMEM((_SL,), jnp.float32),
        pltpu.VMEM((64, 2 * _H), jnp.float32),
        pltpu.SemaphoreType.DMA((_NB,)),
        pltpu.SemaphoreType.DMA((_NB,)),
        pltpu.SemaphoreType.DMA((_NB,)),
    ],
)
def _sc_aggregate(src2_hbm, dst2_hbm, y1_hbm, r1_hbm, cntp_hbm,
                  seg_hbm, cexp_hbm,
                  seg_sh, c_sh, inv_sh, inv_v, src_v, dst_v, rows_v, vals_v,
                  cnt0_v, cnt1_v, invs_v, segb_v, gsem, rsem, vsem):
    cid = lax.axis_index("core")
    sid = lax.axis_index("subcore")
    wid = cid * _NS + sid
    row = sid * _SL

    zb = rows_v.at[0]

    @pl.loop(0, _K)
    def _(r):
        for q in range(_H // _L):
            zb[r, pl.ds(q * _L, _L)] = jnp.zeros((_L,), jnp.float32)

    @pl.loop(0, _SL, step=_K)
    def _(r0):
        pltpu.sync_copy(zb, seg_sh.at[pl.ds(row + r0, _K)])

    @pl.loop(0, _SL, step=_L)
    def _(j):
        invs_v[pl.ds(j, _L)] = jnp.zeros((_L,), jnp.float32)

    pltpu.sync_copy(invs_v, c_sh.at[pl.ds(row, _SL)])

    pltpu.sync_copy(src2_hbm.at[pl.ds(wid * _NCH, _NCH)], src_v)
    pltpu.sync_copy(dst2_hbm.at[pl.ds(wid * _NCH, _NCH)], dst_v)

    pltpu.sync_copy(cntp_hbm.at[0, pl.ds(row, _SL)], cnt0_v)
    pltpu.sync_copy(cntp_hbm.at[1, pl.ds(row, _SL)], cnt1_v)

    @pl.loop(0, _SL, step=_L)
    def _(i):
        a = cnt0_v[pl.ds(i, _L)] + cnt1_v[pl.ds(i, _L)]
        invs_v[pl.ds(i, _L)] = 1.0 / jnp.maximum(a, 1.0)

    pltpu.sync_copy(invs_v, inv_sh.at[pl.ds(row, _SL)])
    plsc.subcore_barrier()

    pltpu.sync_copy(inv_sh, inv_v)

    def _gather(i, b):
        pltpu.async_copy(y1_hbm.at[src_v.at[i]], rows_v.at[b], gsem.at[b])

    def _wait_gather(i, b):
        pltpu.make_async_copy(
            y1_hbm.at[src_v.at[i]], rows_v.at[b], gsem.at[b]).wait()

    def _wait_rscat(i, b):
        pltpu.make_async_copy(
            rows_v.at[b], seg_sh.at[dst_v.at[i]], rsem.at[b]).wait()

    def _wait_vscat(i, b):
        pltpu.make_async_copy(
            vals_v.at[b], c_sh.at[src_v.at[i]], vsem.at[b]).wait()

    def _vals(i, b):
        for j in range(_K // _L):
            iv = dst_v.at[i][pl.ds(j * _L, _L)]
            vals_v[b, pl.ds(j * _L, _L)] = plsc.load_gather(inv_v, [iv])
        pltpu.async_copy(vals_v.at[b], c_sh.at[src_v.at[i]], vsem.at[b],
                         add=True)

    def _rscat(i, b):
        pltpu.async_copy(rows_v.at[b], seg_sh.at[dst_v.at[i]], rsem.at[b],
                         add=True)

    for b in range(_PF):
        _gather(b, b)

    _NG = _NCH // _NB

    @pl.loop(0, _NG)
    def _(g):
        for b in range(_NB):
            i = g * _NB + b
            _wait_gather(i, b)
            _rscat(i, b)

            @pl.when(g > 0)
            def _():
                _wait_vscat(i - _NB, b)

            _vals(i, b)

            b2 = (b + _PF) % _NB
            if b >= _PF:

                @pl.when(i + _PF < _NCH)
                def _():
                    _wait_rscat(i - _PF, b2)
                    _gather(i + _PF, b2)

            else:

                @pl.when(g > 0)
                def _():
                    _wait_rscat(i - _PF, b2)

                @pl.when(i + _PF < _NCH)
                def _():
                    _gather(i + _PF, b2)

    for i in range(_NCH - _NB, _NCH):
        b = i % _NB
        _wait_rscat(i, b)
        _wait_vscat(i, b)

    plsc.subcore_barrier()

    pltpu.sync_copy(c_sh.at[pl.ds(row, _SL)], cnt0_v)

    r1f = jnp.full((_L,), 1.0, jnp.float32) * (cid == 0).astype(jnp.float32)

    @pl.loop(0, _SL, step=128)
    def _(r0):
        rb = rows_v.at[0]
        r1b = rows_v.at[1]
        pltpu.sync_copy(seg_sh.at[pl.ds(row + r0, 128)], rb)
        pltpu.sync_copy(r1_hbm.at[pl.ds(row + r0, 128)], r1b)

        @pl.loop(0, 64)
        def _(pr):
            s0 = plsc.load_gather(
                invs_v, [jnp.full((_L,), r0 + 2 * pr, jnp.int32)])
            s1 = plsc.load_gather(
                invs_v, [jnp.full((_L,), r0 + 2 * pr + 1, jnp.int32)])
            for q in range(_H // _L):
                segb_v[pr, pl.ds(q * _L, _L)] = (
                    rb[2 * pr, pl.ds(q * _L, _L)] * s0
                    + r1b[2 * pr, pl.ds(q * _L, _L)] * r1f)
                segb_v[pr, pl.ds(_H + q * _L, _L)] = (
                    rb[2 * pr + 1, pl.ds(q * _L, _L)] * s1
                    + r1b[2 * pr + 1, pl.ds(q * _L, _L)] * r1f)

        pltpu.sync_copy(segb_v, seg_hbm.at[cid, pl.ds((row + r0) // 2, 64)])

        @pl.loop(0, 64)
        def _(pr):
            c0 = plsc.load_gather(
                cnt0_v, [jnp.full((_L,), r0 + 2 * pr, jnp.int32)])
            c1 = plsc.load_gather(
                cnt0_v, [jnp.full((_L,), r0 + 2 * pr + 1, jnp.int32)])
            for q in range(_H // _L):
                segb_v[pr, pl.ds(q * _L, _L)] = c0
                segb_v[pr, pl.ds(_H + q * _L, _L)] = c1

        pltpu.sync_copy(segb_v, cexp_hbm.at[cid, pl.ds((row + r0) // 2, 64)])


def _tc_pre_body(x_ref, wl_ref, wr_ref, y1_ref, r1_ref):
    y1 = jnp.dot(x_ref[...], wl_ref[...], preferred_element_type=jnp.float32)
    r1 = jnp.dot(x_ref[...], wr_ref[...], preferred_element_type=jnp.float32)
    y1_ref[pl.ds(0, _N), :] = y1
    r1_ref[pl.ds(0, _N), :] = r1
    pad = jnp.zeros((_NP - _N, _H), jnp.float32)
    y1_ref[pl.ds(_N, _NP - _N), :] = pad
    r1_ref[pl.ds(_N, _NP - _N), :] = pad


_tc_pre = pl.pallas_call(
    _tc_pre_body,
    out_shape=(
        jax.ShapeDtypeStruct((_NP, _H), jnp.float32),
        jax.ShapeDtypeStruct((_NP, _H), jnp.float32),
    ),
)


def _tc_post_body(segp_ref, cep_ref,
                  b1_ref, g_ref, bt_ref, w2l_ref, w2r_ref, b2_ref, out_ref):
    def dup(v):
        return jnp.concatenate([v, v], axis=1)

    def fold(v):
        return v[:, :_H] + v[:, _H:]

    mask = (lax.broadcasted_iota(jnp.int32, (_NP // 2, 1), 0)
            < _N // 2).astype(jnp.float32)
    z = segp_ref[0] + segp_ref[1] + dup(b1_ref[...])
    mean = dup(fold(jnp.sum(z * mask, axis=0, keepdims=True))) * (1.0 / _N)
    zc = z - mean
    var = dup(fold(jnp.sum(zc * zc * mask, axis=0, keepdims=True))) * (
        1.0 / _N)
    h = dup(g_ref[...]) * zc * lax.rsqrt(var + 1e-5) + dup(bt_ref[...])
    hm = jnp.maximum(h, 0.0) * mask
    ce = cep_ref[0] + cep_ref[1]
    s1 = fold(jnp.sum(hm * ce, axis=0, keepdims=True)) * (1.0 / _N)
    s2 = fold(jnp.sum(hm, axis=0, keepdims=True)) * (1.0 / _N)
    out_ref[...] = (
        jnp.dot(s1, w2l_ref[...], preferred_element_type=jnp.float32)
        + jnp.dot(s2, w2r_ref[...], preferred_element_type=jnp.float32)
        + b2_ref[...]
    )


_tc_post = pl.pallas_call(
    _tc_post_body,
    out_shape=jax.ShapeDtypeStruct((1, _H), jnp.float32),
)


def kernel(x, edge_index, W1l, b1, W1r, gamma, beta, W2l, b2, W2r):
    cyc = _N + jnp.arange(_EP - _E, dtype=jnp.int32) % (_NP - _N)
    e2 = jnp.concatenate([edge_index, jnp.stack([cyc, cyc])], axis=1)
    src2 = e2[0].reshape(_ER, _K)
    dst2 = e2[1].reshape(_ER, _K)

    y1, r1 = _tc_pre(x, W1l, W1r)
    cntp = _sc_degree(dst2)
    segp, cep = _sc_aggregate(src2, dst2, y1, r1, cntp)

    return _tc_post(
        segp, cep,
        b1[None, :], gamma[None, :], beta[None, :],
        W2l, W2r, b2[None, :],
    )


# speedup vs baseline: 2.3015x; 1.0125x over previous
import functools

import jax
import jax.numpy as jnp
import numpy as np
from jax import lax
from jax.experimental import pallas as pl
from jax.experimental.pallas import tpu as pltpu
from jax.experimental.pallas import tpu_sc as plsc

_N = 10000
_E = 320000
_DIN = 128
_H = 64

_NC = 2
_NS = 16
_L = 16
_NW = _NC * _NS
_NP = 10240
_SL = _NP // _NS
_EPT = _E // _NW
_K = 128
_EP = 2560 * _K
_NCH = _EP // _K // _NW
_ER = _EP // _K
_PF = 2
_NB = 4

_mesh = plsc.VectorSubcoreMesh(core_axis_name="core", subcore_axis_name="subcore")

_EDGE_PAD = np.tile(
    _N + np.arange(_EP - _E, dtype=np.int32) % (_NP - _N), (2, 1))


@functools.partial(
    pl.kernel,
    out_type=jax.ShapeDtypeStruct((_NC, _NP), jnp.float32),
    mesh=_mesh,
    compiler_params=pltpu.CompilerParams(use_tc_tiling_on_sc=False),
    scratch_types=[
        pltpu.VMEM_SHARED((_NP,), jnp.float32),
        pltpu.VMEM((_NCH, _K), jnp.int32),
        pltpu.VMEM((_K,), jnp.float32),
        pltpu.VMEM((_SL,), jnp.float32),
        pltpu.SemaphoreType.DMA,
    ],
)
def _sc_degree(dst2_hbm, cnt_hbm, cnt_sh, idx_v, ones_v, zb_v, sem):
    cid = lax.axis_index("core")
    sid = lax.axis_index("subcore")
    wid = cid * _NS + sid
    row = sid * _SL

    @pl.loop(0, _SL, step=_L)
    def _(j):
        zb_v[pl.ds(j, _L)] = jnp.zeros((_L,), jnp.float32)

    pltpu.sync_copy(zb_v, cnt_sh.at[pl.ds(row, _SL)])
    pltpu.sync_copy(dst2_hbm.at[pl.ds(wid * _NCH, _NCH)], idx_v)

    @pl.loop(0, _K, step=_L)
    def _(j):
        ones_v[pl.ds(j, _L)] = jnp.ones((_L,), jnp.float32)

    plsc.subcore_barrier()

    @pl.loop(0, _NCH, step=5)
    def _(i):
        for k in range(5):
            pltpu.async_copy(ones_v, cnt_sh.at[idx_v.at[i + k]], sem, add=True)
        for k in range(5):
            pltpu.make_async_copy(
                ones_v, cnt_sh.at[idx_v.at[i + k]], sem).wait()

    plsc.subcore_barrier()
    pltpu.sync_copy(cnt_sh.at[pl.ds(row, _SL)], cnt_hbm.at[cid, pl.ds(row, _SL)])


@functools.partial(
    pl.kernel,
    out_type=(
        jax.ShapeDtypeStruct((_NC, _NP // 2, 2 * _H), jnp.float32),
        jax.ShapeDtypeStruct((_NC, _NP // 2, 2 * _H), jnp.float32),
    ),
    mesh=_mesh,
    compiler_params=pltpu.CompilerParams(
        needs_layout_passes=False, use_tc_tiling_on_sc=False),
    scratch_types=[
        pltpu.VMEM_SHARED((_NP, _H), jnp.float32),
        pltpu.VMEM_SHARED((_NP,), jnp.float32),
        pltpu.VMEM_SHARED((_NP,), jnp.float32),
        pltpu.VMEM((_NP,), jnp.float32),
        pltpu.VMEM((_NCH, _K), jnp.int32),
        pltpu.VMEM((_NCH, _K), jnp.int32),
        pltpu.VMEM((_NB, _K, _H), jnp.float32),
        pltpu.VMEM((_NB, _K), jnp.float32),
        pltpu.VMEM((_SL,), jnp.float32),
        pltpu.VMEM((_SL,), jnp.float32),
        pltpu.VMEM((_SL,), jnp.float32),
        pltpu.VMEM((64, 2 * _H), jnp.float32),
        pltpu.SemaphoreType.DMA((_NB,)),
        pltpu.SemaphoreType.DMA((_NB,)),
        pltpu.SemaphoreType.DMA((_NB,)),
    ],
)
def _sc_aggregate(src2_hbm, dst2_hbm, y1_hbm, r1_hbm, cntp_hbm,
                  seg_hbm, cexp_hbm,
                  seg_sh, c_sh, inv_sh, inv_v, src_v, dst_v, rows_v, vals_v,
                  cnt0_v, cnt1_v, invs_v, segb_v, gsem, rsem, vsem):
    cid = lax.axis_index("core")
    sid = lax.axis_index("subcore")
    wid = cid * _NS + sid
    row = sid * _SL

    zb = rows_v.at[0]

    @pl.loop(0, _K)
    def _(r):
        for q in range(_H // _L):
            zb[r, pl.ds(q * _L, _L)] = jnp.zeros((_L,), jnp.float32)

    @pl.loop(0, _SL, step=_K)
    def _(r0):
        pltpu.sync_copy(zb, seg_sh.at[pl.ds(row + r0, _K)])

    @pl.loop(0, _SL, step=_L)
    def _(j):
        invs_v[pl.ds(j, _L)] = jnp.zeros((_L,), jnp.float32)

    pltpu.sync_copy(invs_v, c_sh.at[pl.ds(row, _SL)])

    pltpu.sync_copy(src2_hbm.at[pl.ds(wid * _NCH, _NCH)], src_v)
    pltpu.sync_copy(dst2_hbm.at[pl.ds(wid * _NCH, _NCH)], dst_v)

    pltpu.sync_copy(cntp_hbm.at[0, pl.ds(row, _SL)], cnt0_v)
    pltpu.sync_copy(cntp_hbm.at[1, pl.ds(row, _SL)], cnt1_v)

    @pl.loop(0, _SL, step=_L)
    def _(i):
        a = cnt0_v[pl.ds(i, _L)] + cnt1_v[pl.ds(i, _L)]
        invs_v[pl.ds(i, _L)] = 1.0 / jnp.maximum(a, 1.0)

    pltpu.sync_copy(invs_v, inv_sh.at[pl.ds(row, _SL)])
    plsc.subcore_barrier()

    pltpu.sync_copy(inv_sh, inv_v)

    def _gather(i, b):
        pltpu.async_copy(y1_hbm.at[src_v.at[i]], rows_v.at[b], gsem.at[b])

    def _wait_gather(i, b):
        pltpu.make_async_copy(
            y1_hbm.at[src_v.at[i]], rows_v.at[b], gsem.at[b]).wait()

    def _wait_rscat(i, b):
        pltpu.make_async_copy(
            rows_v.at[b], seg_sh.at[dst_v.at[i]], rsem.at[b]).wait()

    def _wait_vscat(i, b):
        pltpu.make_async_copy(
            vals_v.at[b], c_sh.at[src_v.at[i]], vsem.at[b]).wait()

    def _vals(i, b):
        for j in range(_K // _L):
            iv = dst_v.at[i][pl.ds(j * _L, _L)]
            vals_v[b, pl.ds(j * _L, _L)] = plsc.load_gather(inv_v, [iv])
        pltpu.async_copy(vals_v.at[b], c_sh.at[src_v.at[i]], vsem.at[b],
                         add=True)

    def _rscat(i, b):
        pltpu.async_copy(rows_v.at[b], seg_sh.at[dst_v.at[i]], rsem.at[b],
                         add=True)

    for b in range(_PF):
        _gather(b, b)

    _NG = _NCH // _NB

    @pl.loop(0, _NG)
    def _(g):
        for b in range(_NB):
            i = g * _NB + b
            _wait_gather(i, b)
            _rscat(i, b)

            @pl.when(g > 0)
            def _():
                _wait_vscat(i - _NB, b)

            _vals(i, b)

            b2 = (b + _PF) % _NB
            if b >= _PF:

                @pl.when(i + _PF < _NCH)
                def _():
                    _wait_rscat(i - _PF, b2)
                    _gather(i + _PF, b2)

            else:

                @pl.when(g > 0)
                def _():
                    _wait_rscat(i - _PF, b2)

                @pl.when(i + _PF < _NCH)
                def _():
                    _gather(i + _PF, b2)

    for i in range(_NCH - _NB, _NCH):
        b = i % _NB
        _wait_rscat(i, b)
        _wait_vscat(i, b)

    plsc.subcore_barrier()

    pltpu.sync_copy(c_sh.at[pl.ds(row, _SL)], cnt0_v)

    r1f = jnp.full((_L,), 1.0, jnp.float32) * (cid == 0).astype(jnp.float32)

    @pl.loop(0, _SL, step=128)
    def _(r0):
        rb = rows_v.at[0]
        r1b = rows_v.at[1]
        pltpu.sync_copy(seg_sh.at[pl.ds(row + r0, 128)], rb)
        pltpu.sync_copy(r1_hbm.at[pl.ds(row + r0, 128)], r1b)

        @pl.loop(0, 64)
        def _(pr):
            s0 = plsc.load_gather(
                invs_v, [jnp.full((_L,), r0 + 2 * pr, jnp.int32)])
            s1 = plsc.load_gather(
                invs_v, [jnp.full((_L,), r0 + 2 * pr + 1, jnp.int32)])
            for q in range(_H // _L):
                segb_v[pr, pl.ds(q * _L, _L)] = (
                    rb[2 * pr, pl.ds(q * _L, _L)] * s0
                    + r1b[2 * pr, pl.ds(q * _L, _L)] * r1f)
                segb_v[pr, pl.ds(_H + q * _L, _L)] = (
                    rb[2 * pr + 1, pl.ds(q * _L, _L)] * s1
                    + r1b[2 * pr + 1, pl.ds(q * _L, _L)] * r1f)

        pltpu.sync_copy(segb_v, seg_hbm.at[cid, pl.ds((row + r0) // 2, 64)])

        @pl.loop(0, 64)
        def _(pr):
            c0 = plsc.load_gather(
                cnt0_v, [jnp.full((_L,), r0 + 2 * pr, jnp.int32)])
            c1 = plsc.load_gather(
                cnt0_v, [jnp.full((_L,), r0 + 2 * pr + 1, jnp.int32)])
            for q in range(_H // _L):
                segb_v[pr, pl.ds(q * _L, _L)] = c0
                segb_v[pr, pl.ds(_H + q * _L, _L)] = c1

        pltpu.sync_copy(segb_v, cexp_hbm.at[cid, pl.ds((row + r0) // 2, 64)])


def _tc_pre_body(x_ref, wl_ref, wr_ref, y1_ref, r1_ref):
    y1 = jnp.dot(x_ref[...], wl_ref[...], preferred_element_type=jnp.float32)
    r1 = jnp.dot(x_ref[...], wr_ref[...], preferred_element_type=jnp.float32)
    y1_ref[pl.ds(0, _N), :] = y1
    r1_ref[pl.ds(0, _N), :] = r1
    pad = jnp.zeros((_NP - _N, _H), jnp.float32)
    y1_ref[pl.ds(_N, _NP - _N), :] = pad
    r1_ref[pl.ds(_N, _NP - _N), :] = pad


_tc_pre = pl.pallas_call(
    _tc_pre_body,
    out_shape=(
        jax.ShapeDtypeStruct((_NP, _H), jnp.float32),
        jax.ShapeDtypeStruct((_NP, _H), jnp.float32),
    ),
)


def _tc_post_body(segp_ref, cep_ref,
                  b1_ref, g_ref, bt_ref, w2l_ref, w2r_ref, b2_ref, out_ref):
    def dup(v):
        return jnp.concatenate([v, v], axis=1)

    def fold(v):
        return v[:, :_H] + v[:, _H:]

    mask = (lax.broadcasted_iota(jnp.int32, (_NP // 2, 1), 0)
            < _N // 2).astype(jnp.float32)
    z = segp_ref[0] + segp_ref[1] + dup(b1_ref[...])
    mean = dup(fold(jnp.sum(z * mask, axis=0, keepdims=True))) * (1.0 / _N)
    zc = z - mean
    var = dup(fold(jnp.sum(zc * zc * mask, axis=0, keepdims=True))) * (
        1.0 / _N)
    h = dup(g_ref[...]) * zc * lax.rsqrt(var + 1e-5) + dup(bt_ref[...])
    hm = jnp.maximum(h, 0.0) * mask
    ce = cep_ref[0] + cep_ref[1]
    s1 = fold(jnp.sum(hm * ce, axis=0, keepdims=True)) * (1.0 / _N)
    s2 = fold(jnp.sum(hm, axis=0, keepdims=True)) * (1.0 / _N)
    out_ref[...] = (
        jnp.dot(s1, w2l_ref[...], preferred_element_type=jnp.float32)
        + jnp.dot(s2, w2r_ref[...], preferred_element_type=jnp.float32)
        + b2_ref[...]
    )


_tc_post = pl.pallas_call(
    _tc_post_body,
    out_shape=jax.ShapeDtypeStruct((1, _H), jnp.float32),
)


def kernel(x, edge_index, W1l, b1, W1r, gamma, beta, W2l, b2, W2r):
    e2 = jnp.concatenate([edge_index, _EDGE_PAD], axis=1)
    src2 = e2[0].reshape(_ER, _K)
    dst2 = e2[1].reshape(_ER, _K)

    y1, r1 = _tc_pre(x, W1l, W1r)
    cntp = _sc_degree(dst2)
    segp, cep = _sc_aggregate(src2, dst2, y1, r1, cntp)

    return _tc_post(
        segp, cep,
        b1[None, :], gamma[None, :], beta[None, :],
        W2l, W2r, b2[None, :],
    )
